# baseline (speedup 1.0000x reference)
import jax
import jax.numpy as jnp
from jax.experimental import pallas as pl
from jax.experimental.pallas import tpu as pltpu

_NUM_CLASSES = 1000
_W_ALIGNED = 896
_W_TAIL = _NUM_CLASSES - _W_ALIGNED
_BATCH = 16384
_BLOCK_ROWS = 2048
_NSTEPS = _BATCH // _BLOCK_ROWS


def _onehot_body(x_ref, o_ref, a0, a1, b0, b1, sa0, sa1, sb0, sb1):
    abufs = (a0, a1)
    bbufs = (b0, b1)
    asems = (sa0, sa1)
    bsems = (sb0, sb1)
    i = pl.program_id(0)
    slot = jax.lax.rem(i, 2)

    ids = x_ref[...]
    colsA = jax.lax.broadcasted_iota(
        jnp.int32, (_BLOCK_ROWS, _W_ALIGNED), 1
    )
    valsA = (colsA == ids).astype(jnp.float32)
    colsB = jax.lax.broadcasted_iota(
        jnp.int32, (_BLOCK_ROWS, _W_TAIL), 1
    ) + _W_ALIGNED
    valsB = (colsB == ids).astype(jnp.float32)

    for k in range(2):
        @pl.when(jnp.logical_and(slot == k, i >= 2))
        def _wait_prev(k=k):
            rows = pl.ds((i - 2) * _BLOCK_ROWS, _BLOCK_ROWS)
            pltpu.make_async_copy(
                abufs[k], o_ref.at[rows, pl.ds(0, _W_ALIGNED)], asems[k]
            ).wait()
            pltpu.make_async_copy(
                bbufs[k], o_ref.at[rows, pl.ds(_W_ALIGNED, _W_TAIL)], bsems[k]
            ).wait()

        @pl.when(slot == k)
        def _fill_and_send(k=k):
            rows = pl.ds(i * _BLOCK_ROWS, _BLOCK_ROWS)
            abufs[k][...] = valsA
            pltpu.make_async_copy(
                abufs[k], o_ref.at[rows, pl.ds(0, _W_ALIGNED)], asems[k]
            ).start()
            bbufs[k][...] = valsB
            pltpu.make_async_copy(
                bbufs[k], o_ref.at[rows, pl.ds(_W_ALIGNED, _W_TAIL)], bsems[k]
            ).start()

    @pl.when(i == _NSTEPS - 1)
    def _drain():
        rows = pl.ds(0, _BLOCK_ROWS)
        for k in range(2):
            pltpu.make_async_copy(
                abufs[k], o_ref.at[rows, pl.ds(0, _W_ALIGNED)], asems[k]
            ).wait()
            pltpu.make_async_copy(
                bbufs[k], o_ref.at[rows, pl.ds(_W_ALIGNED, _W_TAIL)], bsems[k]
            ).wait()


def kernel(x1):
    x = x1.astype(jnp.int32).reshape(_BATCH, 1)
    return pl.pallas_call(
        _onehot_body,
        grid=(_NSTEPS,),
        in_specs=[pl.BlockSpec((_BLOCK_ROWS, 1), lambda i: (i, 0))],
        out_specs=pl.BlockSpec(memory_space=pltpu.MemorySpace.HBM),
        out_shape=jax.ShapeDtypeStruct((_BATCH, _NUM_CLASSES), jnp.float32),
        scratch_shapes=(
            [pltpu.VMEM((_BLOCK_ROWS, _W_ALIGNED), jnp.float32)] * 2
            + [pltpu.VMEM((_BLOCK_ROWS, _W_TAIL), jnp.float32)] * 2
            + [pltpu.SemaphoreType.DMA] * 4
        ),
    )(x)
